# baseline (device time: 37037 ns/iter reference)
import jax
import jax.numpy as jnp
from jax import lax
from jax.experimental import pallas as pl
from jax.experimental.pallas import tpu as pltpu

N_DEV = 8


def kernel(x, Win0, Wout0, Win1, Wout1, Win2, Wout2):
    m, d_loc = x.shape
    h_dim = Win0.shape[1]
    chunk = h_dim // N_DEV

    def body(x_ref, win0, win1, win2, wout0, wout1, wout2, out_ref,
             wv, wo, winb, woutb, pbuf, rs_buf, ag_buf, agchunk, xloc,
             send_sems, rs_sems, ag_sems, wsems):
        my = lax.axis_index("i")
        wins_hbm = (win0, win1, win2)
        wouts_hbm = (wout0, wout1, wout2)

        def w_copies(l):
            return (
                pltpu.make_async_copy(wins_hbm[l], wv.at[l % 2],
                                      wsems.at[l, 0]),
                pltpu.make_async_copy(wouts_hbm[l], wo.at[l % 2],
                                      wsems.at[l, 1]),
            )

        for l in (0, 1):
            for c in w_copies(l):
                c.start()

        barrier = pltpu.get_barrier_semaphore()
        for t in range(N_DEV):
            @pl.when(t != my)
            def _():
                pl.semaphore_signal(barrier, inc=1, device_id=(t,),
                                    device_id_type=pl.DeviceIdType.MESH)

        xloc[...] = x_ref[...]
        for l in range(3):
            sl = l % 2
            if l == 0:
                w_copies(0)[0].wait()

            if l == 0:
                for t in range(N_DEV):
                    pt = jnp.dot(xloc[...],
                                 wv.at[sl][:, t * chunk:(t + 1) * chunk],
                                 preferred_element_type=jnp.float32)
                    pbuf[t, :, :] = pt.astype(jnp.bfloat16)
                pl.semaphore_wait(barrier, N_DEV - 1)
            else:
                xb = xloc[...].astype(jnp.bfloat16)

            rs_rdmas = []
            for t in range(N_DEV):
                if l > 0:
                    pt = jnp.dot(xb, winb.at[sl][:, t * chunk:(t + 1) * chunk],
                                 preferred_element_type=jnp.float32)
                    pbuf[t, :, :] = pt.astype(jnp.bfloat16)
                rdma = pltpu.make_async_remote_copy(
                    src_ref=pbuf.at[t],
                    dst_ref=rs_buf.at[my],
                    send_sem=send_sems.at[0, t],
                    recv_sem=rs_sems.at[my],
                    device_id=(t,),
                    device_id_type=pl.DeviceIdType.MESH,
                )
                rs_rdmas.append(rdma)

                @pl.when(t != my)
                def _(rdma=rdma):
                    rdma.start()

            for s in range(N_DEV):
                @pl.when(s != my)
                def _(s=s):
                    pltpu.make_async_remote_copy(
                        src_ref=pbuf.at[s],
                        dst_ref=rs_buf.at[s],
                        send_sem=send_sems.at[0, s],
                        recv_sem=rs_sems.at[s],
                        device_id=(s,),
                        device_id_type=pl.DeviceIdType.MESH,
                    ).wait_recv()

            rs_buf[my, :, :] = pbuf[my, :, :]
            csum = rs_buf[0].astype(jnp.float32)
            for s in range(1, N_DEV):
                csum = csum + rs_buf[s].astype(jnp.float32)
            agchunk[...] = jnp.maximum(csum, 0.0).astype(jnp.bfloat16)

            ag_rdmas = []
            for t in range(N_DEV):
                rdma = pltpu.make_async_remote_copy(
                    src_ref=agchunk,
                    dst_ref=ag_buf.at[my],
                    send_sem=send_sems.at[1, t],
                    recv_sem=ag_sems.at[my],
                    device_id=(t,),
                    device_id_type=pl.DeviceIdType.MESH,
                )
                ag_rdmas.append(rdma)

                @pl.when(t != my)
                def _(rdma=rdma):
                    rdma.start()

            for t in range(N_DEV):
                @pl.when(t != my)
                def _(rdma=rs_rdmas[t]):
                    rdma.wait_send()

            ag_buf[my, :, :] = agchunk[...]

            acc_ref = out_ref if l == 2 else xloc
            if l == 0:
                w_copies(0)[1].wait()
            if l < 2:
                for c in w_copies(l + 1):
                    c.wait()
            nsl = (l + 1) % 2
            wout_l = wo.at[sl] if l == 0 else woutb.at[sl]
            for s in range(N_DEV):
                if l < 2:
                    winb[nsl, :, s * chunk:(s + 1) * chunk] = (
                        wv.at[nsl][:, s * chunk:(s + 1) * chunk]
                        .astype(jnp.bfloat16))
                    woutb[nsl, s * chunk:(s + 1) * chunk, :] = (
                        wo.at[nsl][s * chunk:(s + 1) * chunk, :]
                        .astype(jnp.bfloat16))

                @pl.when(s != my)
                def _(s=s):
                    pltpu.make_async_remote_copy(
                        src_ref=agchunk,
                        dst_ref=ag_buf.at[s],
                        send_sem=send_sems.at[1, s],
                        recv_sem=ag_sems.at[s],
                        device_id=(s,),
                        device_id_type=pl.DeviceIdType.MESH,
                    ).wait_recv()
                lhs = ag_buf[s]
                if l == 0:
                    lhs = lhs.astype(jnp.float32)
                contrib = jnp.dot(lhs,
                                  wout_l[s * chunk:(s + 1) * chunk, :],
                                  preferred_element_type=jnp.float32)
                if s == 0:
                    acc_ref[...] = contrib
                else:
                    acc_ref[...] = acc_ref[...] + contrib

            for t in range(N_DEV):
                @pl.when(t != my)
                def _(rdma=ag_rdmas[t]):
                    rdma.wait_send()

            if l == 0:
                for c in w_copies(2):
                    c.start()

    return pl.pallas_call(
        body,
        out_shape=jax.ShapeDtypeStruct((m, d_loc), jnp.float32),
        in_specs=[pl.BlockSpec(memory_space=pltpu.VMEM)]
        + [pl.BlockSpec(memory_space=pl.ANY)] * 6,
        out_specs=pl.BlockSpec(memory_space=pltpu.VMEM),
        scratch_shapes=[
            pltpu.VMEM((2, d_loc, h_dim), jnp.float32),
            pltpu.VMEM((2, h_dim, d_loc), jnp.float32),
            pltpu.VMEM((2, d_loc, h_dim), jnp.bfloat16),
            pltpu.VMEM((2, h_dim, d_loc), jnp.bfloat16),
            pltpu.VMEM((N_DEV, m, chunk), jnp.bfloat16),
            pltpu.VMEM((N_DEV, m, chunk), jnp.bfloat16),
            pltpu.VMEM((N_DEV, m, chunk), jnp.bfloat16),
            pltpu.VMEM((m, chunk), jnp.bfloat16),
            pltpu.VMEM((m, d_loc), jnp.float32),
            pltpu.SemaphoreType.DMA((2, N_DEV)),
            pltpu.SemaphoreType.DMA((N_DEV,)),
            pltpu.SemaphoreType.DMA((N_DEV,)),
            pltpu.SemaphoreType.DMA((3, 2)),
        ],
        compiler_params=pltpu.CompilerParams(
            collective_id=0,
            vmem_limit_bytes=60 * 1024 * 1024,
        ),
    )(x, Win0, Win1, Win2, Wout0, Wout1, Wout2)
